# baseline (speedup 1.0000x reference)
import dataclasses
import functools

import jax
import jax.numpy as jnp
from jax import lax
from jax.experimental import pallas as pl
from jax.experimental.pallas import tpu as pltpu
from jax.experimental.pallas import tpu_sc as plsc

N_PAD = 100352
NC, NS, L = 2, 16, 16
NW = NC * NS
E_TOTAL = 6400000
EPW = 199936
CE = 4992
NCH = EPW // CE
REM = EPW - NCH * CE
TAIL_BASE = NW * EPW
TAIL_PER_TILE = (E_TOTAL - TAIL_BASE) // L
SLICE = N_PAD // NS


def _sc_compiler_params():
    cp = pltpu.CompilerParams()
    if "needs_layout_passes" in pltpu.CompilerParams.__dataclass_fields__:
        cp = dataclasses.replace(cp, needs_layout_passes=False)
    return cp


def _sc_scatter(edge_index, y):
    mesh = plsc.VectorSubcoreMesh(core_axis_name="c", subcore_axis_name="s")

    @functools.partial(
        pl.kernel,
        compiler_params=_sc_compiler_params(),
        out_type=(
            jax.ShapeDtypeStruct((NW, N_PAD), jnp.float32),
            jax.ShapeDtypeStruct((NW, L), jnp.int32),
        ),
        mesh=mesh,
        scratch_types=[
            pltpu.VMEM((N_PAD,), jnp.float32),
            pltpu.VMEM((2, CE), jnp.int32),
            pltpu.VMEM((CE,), jnp.float32),
            pltpu.VMEM((2, CE), jnp.int32),
            pltpu.VMEM((CE,), jnp.float32),
            pltpu.VMEM((L,), jnp.int32),
            pltpu.SemaphoreType.DMA,
            pltpu.SemaphoreType.DMA,
        ],
    )
    def k(ei_hbm, y_hbm, part_hbm, max_hbm,
          acc, ebuf0, ybuf0, ebuf1, ybuf1,
          maxb, sem0, sem1):
        cid = lax.axis_index("c")
        sid = lax.axis_index("s")
        wid = cid * NS + sid

        ebase = wid * EPW

        def start(c, eb, yb, sem):
            base = pl.multiple_of(ebase + c * CE, 128)
            pltpu.async_copy(ei_hbm.at[:, pl.ds(base, CE)], eb, sem)
            pltpu.async_copy(y_hbm.at[pl.ds(base, CE)], yb, sem)

        def wait(eb, yb, sem):
            pltpu.make_async_copy(ei_hbm.at[:, pl.ds(0, CE)], eb, sem).wait()
            pltpu.make_async_copy(y_hbm.at[pl.ds(0, CE)], yb, sem).wait()

        def scatter_quads(eb, yb, nquads):
            def group(j, mv):
                s = eb[0, pl.ds(j, L)]
                d = eb[1, pl.ds(j, L)]
                yv = yb[pl.ds(j, L)]
                plsc.addupdate_scatter(acc, [d], yv)
                plsc.addupdate_scatter(acc, [s], -yv)
                return jnp.maximum(mv, jnp.maximum(s, d))

            maxb[...] = plsc.parallel_loop(
                0, nquads * 4 * L, step=L, unroll=8, carry=maxb[...])(group)

        assert CE % (4 * L) == 0 and TAIL_PER_TILE % (4 * L) == 0
        assert REM % (4 * L) == 0 and REM % 128 == 0 and NCH % 2 == 0
        start(0, ebuf0, ybuf0, sem0)
        start(1, ebuf1, ybuf1, sem1)

        zero16 = jnp.zeros((L,), jnp.float32)

        @plsc.parallel_loop(0, N_PAD, step=L, unroll=8)
        def _(i):
            acc[pl.ds(i, L)] = zero16

        maxb[...] = jnp.zeros((L,), jnp.int32)

        @pl.loop(0, NCH, step=2)
        def _(c):
            wait(ebuf0, ybuf0, sem0)
            scatter_quads(ebuf0, ybuf0, CE // (4 * L))

            @pl.when(c + 2 < NCH)
            def _():
                start(c + 2, ebuf0, ybuf0, sem0)

            wait(ebuf1, ybuf1, sem1)
            scatter_quads(ebuf1, ybuf1, CE // (4 * L))

            @pl.when(c + 3 < NCH)
            def _():
                start(c + 3, ebuf1, ybuf1, sem1)

        rbase = pl.multiple_of(ebase + NCH * CE, 128)
        pltpu.sync_copy(ei_hbm.at[:, pl.ds(rbase, REM)],
                        ebuf0.at[:, pl.ds(0, REM)])
        pltpu.sync_copy(y_hbm.at[pl.ds(rbase, REM)], ybuf0.at[pl.ds(0, REM)])
        scatter_quads(ebuf0, ybuf0, REM // (4 * L))

        @pl.when(wid < NS)
        def _():
            tbase = pl.multiple_of(TAIL_BASE + wid * TAIL_PER_TILE, 128)
            pltpu.sync_copy(ei_hbm.at[:, pl.ds(tbase, TAIL_PER_TILE)],
                            ebuf0.at[:, pl.ds(0, TAIL_PER_TILE)])
            pltpu.sync_copy(y_hbm.at[pl.ds(tbase, TAIL_PER_TILE)],
                            ybuf0.at[pl.ds(0, TAIL_PER_TILE)])
            scatter_quads(ebuf0, ybuf0, TAIL_PER_TILE // (4 * L))

        pltpu.sync_copy(acc, part_hbm.at[wid])
        pltpu.sync_copy(maxb, max_hbm.at[wid])

    return k(edge_index, y)


def _tc_finalize(partials, maxes):
    rows = N_PAD // 128

    def body(p_ref, m_ref, o_ref, accr):
        w = pl.program_id(0)

        @pl.when(w == 0)
        def _():
            accr[...] = p_ref[0]

        @pl.when(w > 0)
        def _():
            accr[...] += p_ref[0]

        @pl.when(w == NW - 1)
        def _():
            m = jnp.max(m_ref[...])
            o_ref[0, 0] = (jnp.sum(jnp.abs(accr[...]))
                           / (m.astype(jnp.float32) + 1.0))

    p3 = partials.reshape(NW, rows, 128)
    m2 = maxes.reshape(NW * L // 128, 128)
    return pl.pallas_call(
        body,
        grid=(NW,),
        in_specs=[
            pl.BlockSpec((1, rows, 128), lambda w: (w, 0, 0)),
            pl.BlockSpec((NW * L // 128, 128), lambda w: (0, 0)),
        ],
        out_shape=jax.ShapeDtypeStruct((1, 1), jnp.float32),
        out_specs=pl.BlockSpec((1, 1), lambda w: (0, 0),
                               memory_space=pltpu.SMEM),
        scratch_shapes=[pltpu.VMEM((rows, 128), jnp.float32)],
    )(p3, m2)


def kernel(edge_index, y_hat):
    y = y_hat.reshape(-1)
    partials, maxes = _sc_scatter(edge_index, y)
    return _tc_finalize(partials, maxes)[0, 0]


# speedup vs baseline: 1.1531x; 1.1531x over previous
import dataclasses
import functools

import jax
import jax.numpy as jnp
from jax import lax
from jax.experimental import pallas as pl
from jax.experimental.pallas import tpu as pltpu
from jax.experimental.pallas import tpu_sc as plsc

N_PAD = 100352
NC, NS, L = 2, 16, 16
NW = NC * NS
E_TOTAL = 6400000
EPW = 199936
CE = 4992
NCH = EPW // CE
REM = EPW - NCH * CE
TAIL_BASE = NW * EPW
TAIL_PER_TILE = (E_TOTAL - TAIL_BASE) // L
SLICE = N_PAD // NS


def _sc_compiler_params():
    cp = pltpu.CompilerParams()
    if "needs_layout_passes" in pltpu.CompilerParams.__dataclass_fields__:
        cp = dataclasses.replace(cp, needs_layout_passes=False)
    return cp


def _sc_scatter(edge_index, y):
    mesh = plsc.VectorSubcoreMesh(core_axis_name="c", subcore_axis_name="s")

    @functools.partial(
        pl.kernel,
        compiler_params=_sc_compiler_params(),
        out_type=(
            jax.ShapeDtypeStruct((NW, N_PAD), jnp.float32),
            jax.ShapeDtypeStruct((NW, L), jnp.int32),
        ),
        mesh=mesh,
        scratch_types=[
            pltpu.VMEM((N_PAD,), jnp.float32),
            pltpu.VMEM((2, CE), jnp.int32),
            pltpu.VMEM((CE,), jnp.float32),
            pltpu.VMEM((2, CE), jnp.int32),
            pltpu.VMEM((CE,), jnp.float32),
            pltpu.VMEM((L,), jnp.int32),
            pltpu.SemaphoreType.DMA,
            pltpu.SemaphoreType.DMA,
        ],
    )
    def k(ei_hbm, y_hbm, part_hbm, max_hbm,
          acc, ebuf0, ybuf0, ebuf1, ybuf1,
          maxb, sem0, sem1):
        cid = lax.axis_index("c")
        sid = lax.axis_index("s")
        wid = cid * NS + sid

        ebase = wid * EPW

        def start(c, eb, yb, sem):
            base = pl.multiple_of(ebase + c * CE, 128)
            pltpu.async_copy(ei_hbm.at[:, pl.ds(base, CE)], eb, sem)
            pltpu.async_copy(y_hbm.at[pl.ds(base, CE)], yb, sem)

        def wait(eb, yb, sem):
            pltpu.make_async_copy(ei_hbm.at[:, pl.ds(0, CE)], eb, sem).wait()
            pltpu.make_async_copy(y_hbm.at[pl.ds(0, CE)], yb, sem).wait()

        def scatter_quads(eb, yb, nquads):
            def group(j, mv):
                s = eb[0, pl.ds(j, L)]
                d = eb[1, pl.ds(j, L)]
                yv = yb[pl.ds(j, L)]
                plsc.addupdate_scatter(acc, [d], yv)
                plsc.addupdate_scatter(acc, [s], -yv)
                return jnp.maximum(mv, jnp.maximum(s, d))

            maxb[...] = plsc.parallel_loop(
                0, nquads * 4 * L, step=L, unroll=8, carry=maxb[...])(group)

        assert CE % (4 * L) == 0 and TAIL_PER_TILE % (4 * L) == 0
        assert REM % (4 * L) == 0 and REM % 128 == 0 and NCH % 2 == 0
        start(0, ebuf0, ybuf0, sem0)
        start(1, ebuf1, ybuf1, sem1)

        zero16 = jnp.zeros((L,), jnp.float32)

        @plsc.parallel_loop(0, N_PAD, step=L, unroll=8)
        def _(i):
            acc[pl.ds(i, L)] = zero16

        maxb[...] = jnp.zeros((L,), jnp.int32)

        @pl.loop(0, NCH, step=2)
        def _(c):
            wait(ebuf0, ybuf0, sem0)
            scatter_quads(ebuf0, ybuf0, CE // (4 * L))

            @pl.when(c + 2 < NCH)
            def _():
                start(c + 2, ebuf0, ybuf0, sem0)

            wait(ebuf1, ybuf1, sem1)
            scatter_quads(ebuf1, ybuf1, CE // (4 * L))

            @pl.when(c + 3 < NCH)
            def _():
                start(c + 3, ebuf1, ybuf1, sem1)

        rbase = pl.multiple_of(ebase + NCH * CE, 128)
        pltpu.sync_copy(ei_hbm.at[:, pl.ds(rbase, REM)],
                        ebuf0.at[:, pl.ds(0, REM)])
        pltpu.sync_copy(y_hbm.at[pl.ds(rbase, REM)], ybuf0.at[pl.ds(0, REM)])
        scatter_quads(ebuf0, ybuf0, REM // (4 * L))

        @pl.when(wid < NS)
        def _():
            tbase = pl.multiple_of(TAIL_BASE + wid * TAIL_PER_TILE, 128)
            pltpu.sync_copy(ei_hbm.at[:, pl.ds(tbase, TAIL_PER_TILE)],
                            ebuf0.at[:, pl.ds(0, TAIL_PER_TILE)])
            pltpu.sync_copy(y_hbm.at[pl.ds(tbase, TAIL_PER_TILE)],
                            ybuf0.at[pl.ds(0, TAIL_PER_TILE)])
            scatter_quads(ebuf0, ybuf0, TAIL_PER_TILE // (4 * L))

        pltpu.sync_copy(acc, part_hbm.at[wid])
        pltpu.sync_copy(maxb, max_hbm.at[wid])

    return k(edge_index, y)


def _tc_finalize(partials, maxes):
    rows = N_PAD // 128

    def body(p_ref, m_ref, o_ref):
        a = p_ref[...]
        diff = a[:rows]
        for w in range(1, NW):
            diff = diff + a[w * rows:(w + 1) * rows]
        m = jnp.max(m_ref[...])
        o_ref[0, 0] = jnp.sum(jnp.abs(diff)) / (m.astype(jnp.float32) + 1.0)

    p2 = partials.reshape(NW * rows, 128)
    m2 = maxes.reshape(NW * L // 128, 128)
    return pl.pallas_call(
        body,
        out_shape=jax.ShapeDtypeStruct((1, 1), jnp.float32),
        out_specs=pl.BlockSpec(memory_space=pltpu.SMEM),
    )(p2, m2)


def kernel(edge_index, y_hat):
    y = y_hat.reshape(-1)
    partials, maxes = _sc_scatter(edge_index, y)
    return _tc_finalize(partials, maxes)[0, 0]


# speedup vs baseline: 1.1539x; 1.0007x over previous
import dataclasses
import functools

import jax
import jax.numpy as jnp
from jax import lax
from jax.experimental import pallas as pl
from jax.experimental.pallas import tpu as pltpu
from jax.experimental.pallas import tpu_sc as plsc

N_PAD = 100352
NC, NS, L = 2, 16, 16
NW = NC * NS
E_TOTAL = 6400000
EPW = 199936
CE = 4992
NCH = EPW // CE
REM = EPW - NCH * CE
TAIL_BASE = NW * EPW
TAIL_PER_TILE = (E_TOTAL - TAIL_BASE) // L


def _sc_compiler_params():
    cp = pltpu.CompilerParams()
    if "needs_layout_passes" in pltpu.CompilerParams.__dataclass_fields__:
        cp = dataclasses.replace(cp, needs_layout_passes=False)
    return cp


def _sc_scatter(edge_index, y):
    mesh = plsc.VectorSubcoreMesh(core_axis_name="c", subcore_axis_name="s")

    @functools.partial(
        pl.kernel,
        compiler_params=_sc_compiler_params(),
        out_type=(
            jax.ShapeDtypeStruct((NW, N_PAD), jnp.float32),
            jax.ShapeDtypeStruct((NW, L), jnp.int32),
        ),
        mesh=mesh,
        scratch_types=[
            pltpu.VMEM((N_PAD,), jnp.float32),
            pltpu.VMEM((2, CE), jnp.int32),
            pltpu.VMEM((CE,), jnp.float32),
            pltpu.VMEM((2, CE), jnp.int32),
            pltpu.VMEM((CE,), jnp.float32),
            pltpu.VMEM((L,), jnp.int32),
            pltpu.SemaphoreType.DMA,
            pltpu.SemaphoreType.DMA,
        ],
    )
    def k(ei_hbm, y_hbm, part_hbm, max_hbm,
          acc, ebuf0, ybuf0, ebuf1, ybuf1,
          maxb, sem0, sem1):
        cid = lax.axis_index("c")
        sid = lax.axis_index("s")
        wid = cid * NS + sid

        ebase = wid * EPW

        def start(c, eb, yb, sem):
            base = pl.multiple_of(ebase + c * CE, 128)
            pltpu.async_copy(ei_hbm.at[:, pl.ds(base, CE)], eb, sem)
            pltpu.async_copy(y_hbm.at[pl.ds(base, CE)], yb, sem)

        def wait(eb, yb, sem):
            pltpu.make_async_copy(ei_hbm.at[:, pl.ds(0, CE)], eb, sem).wait()
            pltpu.make_async_copy(y_hbm.at[pl.ds(0, CE)], yb, sem).wait()

        def scatter_quads(eb, yb, nquads):
            def group(j, mv):
                s = eb[0, pl.ds(j, L)]
                d = eb[1, pl.ds(j, L)]
                yv = yb[pl.ds(j, L)]
                plsc.addupdate_scatter(acc, [d], yv)
                plsc.addupdate_scatter(acc, [s], -yv)
                return jnp.maximum(mv, jnp.maximum(s, d))

            maxb[...] = plsc.parallel_loop(
                0, nquads * 4 * L, step=L, unroll=8, carry=maxb[...])(group)

        assert CE % (4 * L) == 0 and TAIL_PER_TILE % (4 * L) == 0
        assert REM % (4 * L) == 0 and REM % 128 == 0 and NCH % 2 == 0
        start(0, ebuf0, ybuf0, sem0)
        start(1, ebuf1, ybuf1, sem1)

        zero16 = jnp.zeros((L,), jnp.float32)

        @plsc.parallel_loop(0, N_PAD, step=L, unroll=8)
        def _(i):
            acc[pl.ds(i, L)] = zero16

        maxb[...] = jnp.zeros((L,), jnp.int32)

        @pl.loop(0, NCH, step=2)
        def _(c):
            wait(ebuf0, ybuf0, sem0)
            scatter_quads(ebuf0, ybuf0, CE // (4 * L))

            @pl.when(c + 2 < NCH)
            def _():
                start(c + 2, ebuf0, ybuf0, sem0)

            wait(ebuf1, ybuf1, sem1)
            scatter_quads(ebuf1, ybuf1, CE // (4 * L))

            @pl.when(c + 3 < NCH)
            def _():
                start(c + 3, ebuf1, ybuf1, sem1)

        rbase = pl.multiple_of(ebase + NCH * CE, 128)
        pltpu.sync_copy(ei_hbm.at[:, pl.ds(rbase, REM)],
                        ebuf0.at[:, pl.ds(0, REM)])
        pltpu.sync_copy(y_hbm.at[pl.ds(rbase, REM)], ybuf0.at[pl.ds(0, REM)])
        scatter_quads(ebuf0, ybuf0, REM // (4 * L))

        @pl.when(wid < NS)
        def _():
            tbase = pl.multiple_of(TAIL_BASE + wid * TAIL_PER_TILE, 128)
            pltpu.sync_copy(ei_hbm.at[:, pl.ds(tbase, TAIL_PER_TILE)],
                            ebuf0.at[:, pl.ds(0, TAIL_PER_TILE)])
            pltpu.sync_copy(y_hbm.at[pl.ds(tbase, TAIL_PER_TILE)],
                            ybuf0.at[pl.ds(0, TAIL_PER_TILE)])
            scatter_quads(ebuf0, ybuf0, TAIL_PER_TILE // (4 * L))

        pltpu.sync_copy(acc, part_hbm.at[wid])
        pltpu.sync_copy(maxb, max_hbm.at[wid])

    return k(edge_index, y)


def _tc_finalize(partials, maxes):
    rows = N_PAD // 128

    def body(p_ref, m_ref, o_ref):
        a = p_ref[...]
        diff = a[:rows]
        for w in range(1, NW):
            diff = diff + a[w * rows:(w + 1) * rows]
        m = jnp.max(m_ref[...])
        o_ref[0, 0] = jnp.sum(jnp.abs(diff)) / (m.astype(jnp.float32) + 1.0)

    p2 = partials.reshape(NW * rows, 128)
    m2 = maxes.reshape(NW * L // 128, 128)
    return pl.pallas_call(
        body,
        out_shape=jax.ShapeDtypeStruct((1, 1), jnp.float32),
        out_specs=pl.BlockSpec(memory_space=pltpu.SMEM),
    )(p2, m2)


def kernel(edge_index, y_hat):
    y = y_hat.reshape(-1)
    partials, maxes = _sc_scatter(edge_index, y)
    return _tc_finalize(partials, maxes)[0, 0]
